# baseline (speedup 1.0000x reference)
import functools

import jax
import jax.numpy as jnp
from jax import lax
from jax.experimental import pallas as pl
from jax.experimental.pallas import tpu as pltpu
from jax.experimental.pallas import tpu_sc as plsc

SEQ_LEN = 200
DIM = 32
BATCH = 4096
VOCAB = 1000000
B = BATCH * SEQ_LEN
NC = 2
NS = 16
NW = NC * NS
BQ = BATCH // NW
CB = 8
CHUNK = CB * SEQ_LEN
NCHUNK = BQ // CB
LPR = DIM // 16

_mesh = plsc.VectorSubcoreMesh(core_axis_name="c", subcore_axis_name="s")

NBUCKET = VOCAB // 128
NBFULL = (NBUCKET // NW) * NW
BELEM = 128 * DIM


def _diag_blocks(ib, tb, ncols):
    iota = lax.iota(jnp.int32, 16)
    rows = [iota, iota + 16]
    perms = [(iota + k) & 15 for k in range(16)]
    sconsts = [[perms[k] * DIM + d0 + iota for k in range(16)]
               for d0 in (0, 16)]

    def gcbody(gc, _):
        c0 = gc * 16
        base = c0 * DIM
        for gd in range(2):
            for k in range(16):
                v = plsc.load_gather(ib, [rows[gd], perms[k] + c0])
                plsc.store_scatter(tb, [sconsts[gd][k] + base], v)
        return 0

    lax.fori_loop(0, ncols // 16, gcbody, 0, unroll=2)


@functools.partial(
    pl.kernel,
    mesh=_mesh,
    out_type=jax.ShapeDtypeStruct((VOCAB * DIM,), jnp.float32),
    scratch_types=[
        pltpu.VMEM((DIM, 512), jnp.float32),
        pltpu.VMEM((DIM, 512), jnp.float32),
        pltpu.VMEM((4 * BELEM,), jnp.float32),
        pltpu.VMEM((4 * BELEM,), jnp.float32),
        pltpu.SemaphoreType.DMA,
        pltpu.SemaphoreType.DMA,
        pltpu.SemaphoreType.DMA,
        pltpu.SemaphoreType.DMA,
    ],
    compiler_params=pltpu.CompilerParams(use_tc_tiling_on_sc=True,
                                         needs_layout_passes=False,
                                         disable_bounds_checks=True),
)
def _fmt_kernel(tokt_hbm, tail_hbm, lin_hbm, in0, in1, tb0, tb1, sl0, sl1, ss0, ss1):
    wid = lax.axis_index("s") * NC + lax.axis_index("c")

    def load(q, ib, sl, ncols):
        pltpu.async_copy(tokt_hbm.at[:, pl.ds(q * 512, ncols)],
                         ib.at[:, pl.ds(0, ncols)], sl)

    def transpose(q, ib, tb, sl, ss, ncols):
        pltpu.make_async_copy(tokt_hbm.at[:, pl.ds(q * 512, ncols)],
                              ib.at[:, pl.ds(0, ncols)], sl).wait()

        _diag_blocks(ib, tb, ncols)

        pltpu.async_copy(tb.at[pl.ds(0, ncols * DIM)],
                         lin_hbm.at[pl.ds(q * 512 * DIM, ncols * DIM)], ss)

    def wait_store(q, tb, ss, ncols):
        pltpu.make_async_copy(tb.at[pl.ds(0, ncols * DIM)],
                              lin_hbm.at[pl.ds(q * 512 * DIM, ncols * DIM)],
                              ss).wait()

    ni = NBFULL // (4 * NW)
    load(wid, in0, sl0, 512)

    def pair_body(t, _):
        i = 2 * t
        q0 = wid + i * NW
        q1 = q0 + NW
        q2 = q1 + NW

        @pl.when(i + 1 < ni)
        def _():
            load(q1, in1, sl1, 512)

        @pl.when(i >= 2)
        def _():
            wait_store(q0 - 2 * NW, tb0, ss0, 512)

        transpose(q0, in0, tb0, sl0, ss0, 512)

        @pl.when(i + 2 < ni)
        def _():
            load(q2, in0, sl0, 512)

        @pl.when(i + 1 < ni)
        def _():
            @pl.when(i >= 1)
            def _():
                wait_store(q1 - 2 * NW, tb1, ss1, 512)
            transpose(q1, in1, tb1, sl1, ss1, 512)

        return 0

    lax.fori_loop(0, ni // 2, pair_body, 0)
    qlast = wid + (ni - 1) * NW
    wait_store(qlast - 2 * NW, tb0, ss0, 512)
    transpose(qlast, in0, tb0, sl0, ss0, 512)
    wait_store(qlast - NW, tb1, ss1, 512)
    wait_store(qlast, tb0, ss0, 512)

    @pl.when(wid < NBUCKET - NBFULL)
    def _():
        p = NBFULL + wid
        pltpu.async_copy(tokt_hbm.at[:, pl.ds(p * 128, 128)],
                         in1.at[:, pl.ds(0, 128)], sl1)
        pltpu.make_async_copy(tokt_hbm.at[:, pl.ds(p * 128, 128)],
                              in1.at[:, pl.ds(0, 128)], sl1).wait()
        _diag_blocks(in1, tb1, 128)
        pltpu.sync_copy(tb1.at[pl.ds(0, BELEM)],
                        lin_hbm.at[pl.ds(p * BELEM, BELEM)])

    @pl.when(wid == NBUCKET - NBFULL)
    def _():
        tail_elems = (VOCAB - NBUCKET * 128) * DIM
        pltpu.sync_copy(tail_hbm, tb1.at[pl.ds(0, tail_elems)])
        pltpu.sync_copy(tb1.at[pl.ds(0, tail_elems)],
                        lin_hbm.at[pl.ds(NBUCKET * BELEM, tail_elems)])


CL = 8
NLC = SEQ_LEN // CL


@functools.partial(
    pl.kernel,
    mesh=_mesh,
    out_type=jax.ShapeDtypeStruct((SEQ_LEN, 4, NW, 8, 128), jnp.float32),
    scratch_types=[
        pltpu.VMEM((CL, 128), jnp.int32),
        pltpu.VMEM((CL, 128), jnp.int32),
        pltpu.VMEM((CL, 128, DIM), jnp.float32),
        pltpu.VMEM((CL, 128, DIM), jnp.float32),
        pltpu.VMEM((DIM, 128), jnp.float32),
        pltpu.VMEM((DIM, 128), jnp.float32),
        pltpu.VMEM((SEQ_LEN, DIM), jnp.float32),
        pltpu.SemaphoreType.DMA,
        pltpu.SemaphoreType.DMA,
        pltpu.SemaphoreType.DMA,
        pltpu.SemaphoreType.DMA,
    ],
    compiler_params=pltpu.CompilerParams(use_tc_tiling_on_sc=False,
                                         needs_layout_passes=False,
                                         disable_bounds_checks=True),
)
def _emb_kernel(idxt_hbm, tok_hbm, pos_hbm, out_hbm,
                idx0, idx1, rows0, rows1, tb0, tb1, pos_v,
                sg0, sg1, st0, st1):
    wid = lax.axis_index("s") * NC + lax.axis_index("c")
    pltpu.sync_copy(pos_hbm, pos_v)

    iota = lax.iota(jnp.int32, 16)
    perms = [(iota + k) & 15 for k in range(16)]
    tbs = [tb0, tb1]
    sts = [st0, st1]

    def drain_tiles(p):
        for t in range(4):
            pltpu.make_async_copy(tbs[p].at[pl.ds(8 * t, 8)],
                                  out_hbm.at[0, t, wid], sts[p]).wait()

    def launch(c, iv, rv, sg):
        @pl.when(c < NLC)
        def _():
            pltpu.sync_copy(
                idxt_hbm.at[pl.ds(c * CL, CL), pl.ds(128 * wid, 128)], iv)
            for li in range(CL):
                pltpu.async_copy(tok_hbm.at[iv.at[li]], rv.at[li], sg)

    def process(c, iv, rv, sg):
        for li in range(CL):
            pltpu.make_async_copy(tok_hbm.at[iv.at[li]], rv.at[li],
                                  sg).wait()

        def li_pair(li2, _):
            for p in range(2):
                li = 2 * li2 + p
                drain_tiles(p)
                l = c * CL + li
                src = rv.at[li]
                for d0 in (0, 16):
                    pvecs = [plsc.load_gather(
                        pos_v, [jnp.full((16,), l, jnp.int32),
                                perms[k] + d0]) for k in range(16)]

                    def bgroup(bg, _):
                        rowsv = iota + bg * 16
                        for k in range(16):
                            v = plsc.load_gather(
                                src, [rowsv, perms[k] + d0])
                            plsc.store_scatter(
                                tbs[p], [perms[k] + d0, rowsv],
                                v + pvecs[k])
                        return 0

                    lax.fori_loop(0, 8, bgroup, 0, unroll=2)
                for t in range(4):
                    pltpu.async_copy(tbs[p].at[pl.ds(8 * t, 8)],
                                     out_hbm.at[l, t, wid], sts[p])
            return 0

        lax.fori_loop(0, CL // 2, li_pair, 0)

    for p in range(2):
        for t in range(4):
            pltpu.async_copy(tbs[p].at[pl.ds(8 * t, 8)],
                             out_hbm.at[0, t, wid], sts[p])

    launch(0, idx0, rows0, sg0)
    launch(1, idx1, rows1, sg1)
    process(0, idx0, rows0, sg0)

    def pair_body(u, _):
        c = 2 * u + 1
        launch(c + 1, idx0, rows0, sg0)
        process(c, idx1, rows1, sg1)
        launch(c + 2, idx1, rows1, sg1)
        process(c + 1, idx0, rows0, sg0)
        return 0

    lax.fori_loop(0, (NLC - 1) // 2, pair_body, 0)
    drain_tiles(0)
    drain_tiles(1)


def kernel(inputs, token_table, position_table):
    tail = token_table[NBUCKET * 128:].reshape((VOCAB - NBUCKET * 128) * DIM)
    tok = _fmt_kernel(token_table.T, tail).reshape(VOCAB, DIM)
    out5 = _emb_kernel(inputs.T.astype(jnp.int32), tok, position_table)
    return out5.transpose(2, 4, 0, 1, 3).reshape(BATCH, SEQ_LEN, DIM)


# speedup vs baseline: 1.5350x; 1.5350x over previous
import functools

import jax
import jax.numpy as jnp
from jax import lax
from jax.experimental import pallas as pl
from jax.experimental.pallas import tpu as pltpu
from jax.experimental.pallas import tpu_sc as plsc

SEQ_LEN = 200
DIM = 32
BATCH = 4096
VOCAB = 1000000
B = BATCH * SEQ_LEN
NC = 2
NS = 16
NW = NC * NS
BQ = BATCH // NW
CB = 8
CHUNK = CB * SEQ_LEN
NCHUNK = BQ // CB
LPR = DIM // 16

_mesh = plsc.VectorSubcoreMesh(core_axis_name="c", subcore_axis_name="s")

NBUCKET = VOCAB // 128
NBFULL = (NBUCKET // NW) * NW
BELEM = 128 * DIM


def _diag_blocks(ib, tb, ncols):
    iota = lax.iota(jnp.int32, 16)
    rows = [iota, iota + 16]
    perms = [(iota + k) & 15 for k in range(16)]
    sconsts = [[perms[k] * DIM + d0 + iota for k in range(16)]
               for d0 in (0, 16)]

    @plsc.parallel_loop(0, ncols // 16)
    def _(gc):
        c0 = gc * 16
        base = c0 * DIM
        for gd in range(2):
            for k in range(16):
                v = plsc.load_gather(ib, [rows[gd], perms[k] + c0])
                plsc.store_scatter(tb, [sconsts[gd][k] + base], v)


@functools.partial(
    pl.kernel,
    mesh=_mesh,
    out_type=jax.ShapeDtypeStruct((VOCAB * DIM,), jnp.float32),
    scratch_types=[
        pltpu.VMEM((DIM, 512), jnp.float32),
        pltpu.VMEM((DIM, 512), jnp.float32),
        pltpu.VMEM((4 * BELEM,), jnp.float32),
        pltpu.VMEM((4 * BELEM,), jnp.float32),
        pltpu.SemaphoreType.DMA,
        pltpu.SemaphoreType.DMA,
        pltpu.SemaphoreType.DMA,
        pltpu.SemaphoreType.DMA,
    ],
    compiler_params=pltpu.CompilerParams(use_tc_tiling_on_sc=True,
                                         needs_layout_passes=False,
                                         disable_bounds_checks=True),
)
def _fmt_kernel(tokt_hbm, tail_hbm, lin_hbm, in0, in1, tb0, tb1, sl0, sl1, ss0, ss1):
    wid = lax.axis_index("s") * NC + lax.axis_index("c")

    def load(q, ib, sl, ncols):
        pltpu.async_copy(tokt_hbm.at[:, pl.ds(q * 512, ncols)],
                         ib.at[:, pl.ds(0, ncols)], sl)

    def transpose(q, ib, tb, sl, ss, ncols):
        pltpu.make_async_copy(tokt_hbm.at[:, pl.ds(q * 512, ncols)],
                              ib.at[:, pl.ds(0, ncols)], sl).wait()

        _diag_blocks(ib, tb, ncols)

        pltpu.async_copy(tb.at[pl.ds(0, ncols * DIM)],
                         lin_hbm.at[pl.ds(q * 512 * DIM, ncols * DIM)], ss)

    def wait_store(q, tb, ss, ncols):
        pltpu.make_async_copy(tb.at[pl.ds(0, ncols * DIM)],
                              lin_hbm.at[pl.ds(q * 512 * DIM, ncols * DIM)],
                              ss).wait()

    ni = NBFULL // (4 * NW)
    load(wid, in0, sl0, 512)

    def pair_body(t, _):
        i = 2 * t
        q0 = wid + i * NW
        q1 = q0 + NW
        q2 = q1 + NW

        @pl.when(i + 1 < ni)
        def _():
            load(q1, in1, sl1, 512)

        @pl.when(i >= 2)
        def _():
            wait_store(q0 - 2 * NW, tb0, ss0, 512)

        transpose(q0, in0, tb0, sl0, ss0, 512)

        @pl.when(i + 2 < ni)
        def _():
            load(q2, in0, sl0, 512)

        @pl.when(i + 1 < ni)
        def _():
            @pl.when(i >= 1)
            def _():
                wait_store(q1 - 2 * NW, tb1, ss1, 512)
            transpose(q1, in1, tb1, sl1, ss1, 512)

        return 0

    lax.fori_loop(0, ni // 2, pair_body, 0)
    qlast = wid + (ni - 1) * NW
    wait_store(qlast - 2 * NW, tb0, ss0, 512)
    transpose(qlast, in0, tb0, sl0, ss0, 512)
    wait_store(qlast - NW, tb1, ss1, 512)
    wait_store(qlast, tb0, ss0, 512)

    @pl.when(wid < NBUCKET - NBFULL)
    def _():
        p = NBFULL + wid
        pltpu.async_copy(tokt_hbm.at[:, pl.ds(p * 128, 128)],
                         in1.at[:, pl.ds(0, 128)], sl1)
        pltpu.make_async_copy(tokt_hbm.at[:, pl.ds(p * 128, 128)],
                              in1.at[:, pl.ds(0, 128)], sl1).wait()
        _diag_blocks(in1, tb1, 128)
        pltpu.sync_copy(tb1.at[pl.ds(0, BELEM)],
                        lin_hbm.at[pl.ds(p * BELEM, BELEM)])

    @pl.when(wid == NBUCKET - NBFULL)
    def _():
        tail_elems = (VOCAB - NBUCKET * 128) * DIM
        pltpu.sync_copy(tail_hbm, tb1.at[pl.ds(0, tail_elems)])
        pltpu.sync_copy(tb1.at[pl.ds(0, tail_elems)],
                        lin_hbm.at[pl.ds(NBUCKET * BELEM, tail_elems)])


CL = 8
NLC = SEQ_LEN // CL


@functools.partial(
    pl.kernel,
    mesh=_mesh,
    out_type=jax.ShapeDtypeStruct((SEQ_LEN, 4, NW, 8, 128), jnp.float32),
    scratch_types=[
        pltpu.VMEM((CL, 128), jnp.int32),
        pltpu.VMEM((CL, 128), jnp.int32),
        pltpu.VMEM((CL, 128, DIM), jnp.float32),
        pltpu.VMEM((CL, 128, DIM), jnp.float32),
        pltpu.VMEM((DIM, 128), jnp.float32),
        pltpu.VMEM((DIM, 128), jnp.float32),
        pltpu.VMEM((SEQ_LEN, DIM), jnp.float32),
        pltpu.SemaphoreType.DMA,
        pltpu.SemaphoreType.DMA,
        pltpu.SemaphoreType.DMA,
        pltpu.SemaphoreType.DMA,
    ],
    compiler_params=pltpu.CompilerParams(use_tc_tiling_on_sc=False,
                                         needs_layout_passes=False,
                                         disable_bounds_checks=True),
)
def _emb_kernel(idxt_hbm, tok_hbm, pos_hbm, out_hbm,
                idx0, idx1, rows0, rows1, tb0, tb1, pos_v,
                sg0, sg1, st0, st1):
    wid = lax.axis_index("s") * NC + lax.axis_index("c")
    pltpu.sync_copy(pos_hbm, pos_v)

    iota = lax.iota(jnp.int32, 16)
    perms = [(iota + k) & 15 for k in range(16)]
    tbs = [tb0, tb1]
    sts = [st0, st1]

    def drain_tiles(p):
        for t in range(4):
            pltpu.make_async_copy(tbs[p].at[pl.ds(8 * t, 8)],
                                  out_hbm.at[0, t, wid], sts[p]).wait()

    def launch(c, iv, rv, sg):
        @pl.when(c < NLC)
        def _():
            pltpu.sync_copy(
                idxt_hbm.at[pl.ds(c * CL, CL), pl.ds(128 * wid, 128)], iv)
            for li in range(CL):
                pltpu.async_copy(tok_hbm.at[iv.at[li]], rv.at[li], sg)

    def process(c, iv, rv, sg):
        for li in range(CL):
            pltpu.make_async_copy(tok_hbm.at[iv.at[li]], rv.at[li],
                                  sg).wait()

        def li_pair(li2, _):
            for p in range(2):
                li = 2 * li2 + p
                drain_tiles(p)
                l = c * CL + li
                src = rv.at[li]
                for d0 in (0, 16):
                    pvecs = [plsc.load_gather(
                        pos_v, [jnp.full((16,), l, jnp.int32),
                                perms[k] + d0]) for k in range(16)]

                    @plsc.parallel_loop(0, 8)
                    def _(bg):
                        rowsv = iota + bg * 16
                        for k in range(16):
                            v = plsc.load_gather(
                                src, [rowsv, perms[k] + d0])
                            plsc.store_scatter(
                                tbs[p], [perms[k] + d0, rowsv],
                                v + pvecs[k])
                for t in range(4):
                    pltpu.async_copy(tbs[p].at[pl.ds(8 * t, 8)],
                                     out_hbm.at[l, t, wid], sts[p])
            return 0

        lax.fori_loop(0, CL // 2, li_pair, 0)

    for p in range(2):
        for t in range(4):
            pltpu.async_copy(tbs[p].at[pl.ds(8 * t, 8)],
                             out_hbm.at[0, t, wid], sts[p])

    launch(0, idx0, rows0, sg0)
    launch(1, idx1, rows1, sg1)
    process(0, idx0, rows0, sg0)

    def pair_body(u, _):
        c = 2 * u + 1
        launch(c + 1, idx0, rows0, sg0)
        process(c, idx1, rows1, sg1)
        launch(c + 2, idx1, rows1, sg1)
        process(c + 1, idx0, rows0, sg0)
        return 0

    lax.fori_loop(0, (NLC - 1) // 2, pair_body, 0)
    drain_tiles(0)
    drain_tiles(1)


def kernel(inputs, token_table, position_table):
    tail = token_table[NBUCKET * 128:].reshape((VOCAB - NBUCKET * 128) * DIM)
    tok = _fmt_kernel(token_table.T, tail).reshape(VOCAB, DIM)
    out5 = _emb_kernel(inputs.T.astype(jnp.int32), tok, position_table)
    return out5.transpose(2, 4, 0, 1, 3).reshape(BATCH, SEQ_LEN, DIM)
